# Initial kernel scaffold; baseline (speedup 1.0000x reference)
import jax
import jax.numpy as jnp
from jax.experimental import pallas as pl


def kernel(x, duty_cycles):
    raise NotImplementedError("write your pallas kernel here")


# speedup vs baseline: 10.7903x; 10.7903x over previous
import functools

import jax
import jax.numpy as jnp
from jax.experimental import pallas as pl


def _kwinners_block(x_ref, duty_ref, o_ref, *, k):
    x = x_ref[...]
    factor = jnp.exp(-duty_ref[...])
    boosted = x * factor

    u = jax.lax.bitcast_convert_type(boosted, jnp.uint32)
    neg = u >= jnp.uint32(0x80000000)
    ukey = jnp.where(neg, ~u, u | jnp.uint32(0x80000000))

    rows = x.shape[0]
    kk = jnp.int32(k)

    def step(i, cand):
        b = (jnp.int32(31) - i).astype(jnp.uint32)
        trial = cand | (jnp.uint32(1) << b)
        cnt = jnp.sum((ukey >= trial).astype(jnp.int32), axis=1, keepdims=True)
        return jnp.where(cnt >= kk, trial, cand)

    cand0 = jnp.zeros((rows, 1), dtype=jnp.uint32)
    thresh = jax.lax.fori_loop(0, 32, step, cand0)

    o_ref[...] = jnp.where(ukey < thresh, jnp.zeros_like(x), x)


def kernel(x, duty_cycles):
    batch, n = x.shape
    k = int(round(n * 0.25))
    rows_per_block = 8
    grid = batch // rows_per_block
    duty2 = duty_cycles.reshape(1, n)

    return pl.pallas_call(
        functools.partial(_kwinners_block, k=k),
        grid=(grid,),
        in_specs=[
            pl.BlockSpec((rows_per_block, n), lambda i: (i, 0)),
            pl.BlockSpec((1, n), lambda i: (0, 0)),
        ],
        out_specs=pl.BlockSpec((rows_per_block, n), lambda i: (i, 0)),
        out_shape=jax.ShapeDtypeStruct((batch, n), x.dtype),
    )(x, duty2)
